# baseline (speedup 1.0000x reference)
import functools

import jax
import jax.numpy as jnp
from jax import lax
from jax.experimental import pallas as pl
from jax.experimental.pallas import tpu as pltpu
from jax.experimental.pallas import tpu_sc as plsc

B = 4096
D = 64

_NC = 2
_NS = 16
_NW = _NC * _NS
_BPW = B // _NW

@functools.cache
def _make_sc_gather():
    mesh = plsc.VectorSubcoreMesh(core_axis_name="c", subcore_axis_name="s")

    @functools.partial(
        pl.kernel,
        mesh=mesh,
        out_type=[
            jax.ShapeDtypeStruct((B, D), jnp.float32),
            jax.ShapeDtypeStruct((B, D), jnp.float32),
        ],
        scratch_types=[
            pltpu.VMEM((_BPW,), jnp.int32),
            pltpu.VMEM((_BPW,), jnp.int32),
            pltpu.VMEM((_BPW, D), jnp.float32),
            pltpu.VMEM((_BPW, D), jnp.float32),
            pltpu.SemaphoreType.DMA,
            pltpu.SemaphoreType.DMA,
        ],
        compiler_params=pltpu.CompilerParams(use_tc_tiling_on_sc=False),
    )
    def _sc_gather(uidx_hbm, iidx_hbm, utab_hbm, itab_hbm, uout_hbm, iout_hbm,
                   uidx_v, iidx_v, urows_v, irows_v, usem, isem):
        wid = lax.axis_index("s") * _NC + lax.axis_index("c")
        base = wid * _BPW
        pltpu.sync_copy(uidx_hbm.at[pl.ds(base, _BPW)], uidx_v)
        pltpu.sync_copy(iidx_hbm.at[pl.ds(base, _BPW)], iidx_v)
        ucp = pltpu.async_copy(utab_hbm.at[uidx_v], urows_v, usem)
        icp = pltpu.async_copy(itab_hbm.at[iidx_v], irows_v, isem)
        ucp.wait()
        icp.wait()
        pltpu.sync_copy(urows_v, uout_hbm.at[pl.ds(base, _BPW)])
        pltpu.sync_copy(irows_v, iout_hbm.at[pl.ds(base, _BPW)])

    return _sc_gather


_BM = 512
_BN = 1024


def _mm_body(u_ref, i_ref, o_ref):
    o_ref[...] = lax.dot_general(
        u_ref[...], i_ref[...],
        (((1,), (1,)), ((), ())),
        preferred_element_type=jnp.float32,
    )


_matmul = pl.pallas_call(
    _mm_body,
    grid=(B // _BM, B // _BN),
    in_specs=[
        pl.BlockSpec((_BM, D), lambda i, j: (i, 0)),
        pl.BlockSpec((_BN, D), lambda i, j: (j, 0)),
    ],
    out_specs=pl.BlockSpec((_BM, _BN), lambda i, j: (i, j)),
    out_shape=jax.ShapeDtypeStruct((B, B), jnp.float32),
)


@jax.jit
def kernel(user_indices, item_indices, user_table, item_table):
    user_embs, item_embs = _make_sc_gather()(
        user_indices.astype(jnp.int32), item_indices.astype(jnp.int32),
        user_table, item_table)
    return _matmul(user_embs, item_embs)
